# baseline (speedup 1.0000x reference)
import functools

import jax
import jax.numpy as jnp
import numpy as np
from jax import lax
from jax.experimental import pallas as pl
from jax.experimental.pallas import tpu as pltpu
from jax.experimental.pallas import tpu_sc as plsc

_ROWS, _COLS = 4096, 768
_SIZE = 3

_key = jax.random.key(42)
_k1, _k2 = jax.random.split(_key)
_R_IDX = int(jax.random.randint(_k1, (), 0, _ROWS - _SIZE))
_PERM = [int(v) for v in np.asarray(jax.random.permutation(_k2, _SIZE))]

_NC, _NS = 2, 16
_NW = _NC * _NS
_RPW = _ROWS // _NW

_OWNERS: dict[int, list[int]] = {}
for _j in range(_SIZE):
    _OWNERS.setdefault((_R_IDX + _j) // _RPW, []).append(_j)

_mesh = plsc.VectorSubcoreMesh(
    core_axis_name="c", subcore_axis_name="s",
    num_cores=_NC, num_subcores=_NS)


_NCHUNK = 4
_CH = _RPW // _NCHUNK


@functools.partial(
    pl.kernel,
    out_type=jax.ShapeDtypeStruct((_ROWS, _COLS), jnp.float32),
    mesh=_mesh,
    scratch_types=[pltpu.VMEM_SHARED((_NS * _RPW, _COLS), jnp.float32)]
    + [pltpu.SemaphoreType.DMA] * (2 * _NCHUNK),
)
def _sc_permute(x_hbm, o_hbm, shared, *sems):
    in_sems, out_sems = sems[:_NCHUNK], sems[_NCHUNK:]
    sid = lax.axis_index("s")
    wid = sid * _NC + lax.axis_index("c")
    base = wid * _RPW
    slab = shared.at[pl.ds(sid * _RPW, _RPW)]
    loads = [
        pltpu.make_async_copy(
            x_hbm.at[pl.ds(base + k * _CH, _CH)],
            slab.at[pl.ds(k * _CH, _CH)], in_sems[k])
        for k in range(_NCHUNK)
    ]
    stores = [
        pltpu.make_async_copy(
            slab.at[pl.ds(k * _CH, _CH)],
            o_hbm.at[pl.ds(base + k * _CH, _CH)], out_sems[k])
        for k in range(_NCHUNK)
    ]
    for c in loads:
        c.start()
    for k in range(_NCHUNK):
        loads[k].wait()
        for _owner, _js in _OWNERS.items():
            _rel = [j for j in _js
                    if (_R_IDX + j) // _CH - _owner * _NCHUNK == k]
            if not _rel:
                continue

            @pl.when(wid == _owner)
            def _(_owner=_owner, _rel=_rel):
                for j in _rel:
                    pltpu.sync_copy(
                        x_hbm.at[pl.ds(_R_IDX + _PERM[j], 1)],
                        slab.at[pl.ds(_R_IDX - _owner * _RPW + j, 1)])
        stores[k].start()
    for c in stores:
        c.wait()


def kernel(x, y):
    return (_sc_permute(x), y)


# speedup vs baseline: 1.0026x; 1.0026x over previous
import functools

import jax
import jax.numpy as jnp
import numpy as np
from jax import lax
from jax.experimental import pallas as pl
from jax.experimental.pallas import tpu as pltpu
from jax.experimental.pallas import tpu_sc as plsc

_ROWS, _COLS = 4096, 768
_SIZE = 3


def _draw_constants():
    key = jax.random.key(42)
    k1, k2 = jax.random.split(key)
    r = int(jax.random.randint(k1, (), 0, _ROWS - _SIZE))
    p = [int(v) for v in np.asarray(jax.random.permutation(k2, _SIZE))]
    return r, p


try:
    with jax.default_device(jax.local_devices(backend="cpu")[0]):
        _R_IDX, _PERM = _draw_constants()
except Exception:
    _R_IDX, _PERM = _draw_constants()

_NC, _NS = 2, 16
_NW = _NC * _NS
_RPW = _ROWS // _NW

_OWNERS: dict[int, list[int]] = {}
for _j in range(_SIZE):
    _OWNERS.setdefault((_R_IDX + _j) // _RPW, []).append(_j)

_NCHUNK = 4
_CH = _RPW // _NCHUNK


def _sc_permute(x_hbm, o_hbm, slab, *sems):
    in_sems, out_sems = sems[:_NCHUNK], sems[_NCHUNK:]
    wid = lax.axis_index("s") * _NC + lax.axis_index("c")
    base = wid * _RPW
    loads = [
        pltpu.make_async_copy(
            x_hbm.at[pl.ds(base + k * _CH, _CH)],
            slab.at[pl.ds(k * _CH, _CH)], in_sems[k])
        for k in range(_NCHUNK)
    ]
    stores = [
        pltpu.make_async_copy(
            slab.at[pl.ds(k * _CH, _CH)],
            o_hbm.at[pl.ds(base + k * _CH, _CH)], out_sems[k])
        for k in range(_NCHUNK)
    ]
    for c in loads:
        c.start()
    for k in range(_NCHUNK):
        loads[k].wait()
        for _owner, _js in _OWNERS.items():
            _rel = [j for j in _js
                    if (_R_IDX + j) // _CH - _owner * _NCHUNK == k]
            if not _rel:
                continue

            @pl.when(wid == _owner)
            def _(_owner=_owner, _rel=_rel):
                for j in _rel:
                    pltpu.sync_copy(
                        x_hbm.at[pl.ds(_R_IDX + _PERM[j], 1)],
                        slab.at[pl.ds(_R_IDX - _owner * _RPW + j, 1)])
        stores[k].start()
    for c in stores:
        c.wait()


@functools.cache
def _build():
    mesh = plsc.VectorSubcoreMesh(
        core_axis_name="c", subcore_axis_name="s",
        num_cores=_NC, num_subcores=_NS)
    return pl.kernel(
        _sc_permute,
        out_type=jax.ShapeDtypeStruct((_ROWS, _COLS), jnp.float32),
        mesh=mesh,
        scratch_types=[pltpu.VMEM((_RPW, _COLS), jnp.float32)]
        + [pltpu.SemaphoreType.DMA] * (2 * _NCHUNK),
    )


def kernel(x, y):
    return (_build()(x), y)
